# Initial kernel scaffold; baseline (speedup 1.0000x reference)
import jax
import jax.numpy as jnp
from jax.experimental import pallas as pl


def kernel(patches):
    raise NotImplementedError("write your pallas kernel here")


# speedup vs baseline: 59.3698x; 59.3698x over previous
import functools

import jax
import jax.numpy as jnp
from jax import lax
from jax.experimental import pallas as pl
from jax.experimental.pallas import tpu as pltpu
from jax.experimental.pallas import tpu_sc as plsc

T = 320
B = 128
C = 768
REMAIN_T = 159
NC, NS, L = 2, 16, 16
NW = NC * NS
ROWS = 64
NG = REMAIN_T * 2
KMAX = (NG + NW - 1) // NW
FWD_PER_W = 16
NWF = T // FWD_PER_W


def _shuffle_sc(p2, lines_t):
    mesh = plsc.VectorSubcoreMesh(core_axis_name="c", subcore_axis_name="s")

    @functools.partial(
        pl.kernel,
        mesh=mesh,
        out_type=[
            jax.ShapeDtypeStruct((REMAIN_T * B, C), jnp.float32),
            jax.ShapeDtypeStruct((T, B), jnp.int32),
            jax.ShapeDtypeStruct((T, B), jnp.int32),
        ],
        scratch_types=[
            pltpu.VMEM((16, B), jnp.int32),
            pltpu.VMEM((ROWS,), jnp.int32),
            pltpu.VMEM((ROWS, C), jnp.float32),
            pltpu.VMEM((FWD_PER_W, B), jnp.int32),
            pltpu.VMEM((FWD_PER_W, B), jnp.int32),
            pltpu.SemaphoreType.DMA,
        ],
    )
    def k(p2_hbm, lines_hbm, out_hbm, fwd_hbm, bwd_hbm,
          lines_v, idx_v, buf, stf, stb, sem):
        wid = lax.axis_index("s") * NC + lax.axis_index("c")
        iota = lax.iota(jnp.int32, L)
        zeros = jnp.zeros((L,), jnp.int32)

        pltpu.sync_copy(lines_hbm, lines_v)

        @pl.when(wid < NWF)
        def _():
            k0 = wid * FWD_PER_W
            for c in range(B // L):
                p_lo = zeros
                p_hi = zeros
                for j in range(16):
                    lv = lines_v[j, pl.ds(c * L, L)]
                    amt = (lv & 7) << 2
                    sh = jnp.full((L,), j, jnp.int32) << amt
                    lo = lv < 8
                    p_lo = p_lo + jnp.where(lo, sh, zeros)
                    p_hi = p_hi + jnp.where(lo, zeros, sh)
                for rr in range(FWD_PER_W):
                    krow = k0 + rr
                    jf = krow // 20
                    i_f = krow % 20
                    p = p_lo if rr < 8 else p_hi
                    inv_vec = lax.shift_right_logical(
                        p, jnp.int32(4 * (rr & 7))) & 15
                    stf[rr, pl.ds(c * L, L)] = (
                        16 * i_f + lines_v[jf, pl.ds(c * L, L)])
                    stb[rr, pl.ds(c * L, L)] = 20 * inv_vec + wid
            pltpu.sync_copy(stf, fwd_hbm.at[pl.ds(k0, FWD_PER_W)])
            pltpu.sync_copy(stb, bwd_hbm.at[pl.ds(k0, FWD_PER_W)])

        def do_chunk(g):
            t = g // 2
            b0 = (g % 2) * ROWS
            jf = t // 20
            i_f = t % 20
            for c in range(ROWS // L):
                lvec = lines_v[jf, pl.ds(b0 + c * L, L)]
                bvec = b0 + c * L + iota
                idx_v[pl.ds(c * L, L)] = 2048 * i_f + 128 * lvec + bvec
            pltpu.async_copy(p2_hbm.at[idx_v], buf, sem).wait()
            pltpu.sync_copy(buf, out_hbm.at[pl.ds(g * ROWS, ROWS)])

        for kk in range(KMAX - 1):
            do_chunk(wid + NW * kk)
        g_last = wid + NW * (KMAX - 1)

        @pl.when(g_last < NG)
        def _():
            do_chunk(g_last)

    return k(p2, lines_t)


def kernel(patches):
    t, b, c = patches.shape
    keys = jax.random.split(jax.random.key(42), b)
    lines = jax.vmap(lambda kk: jax.random.permutation(kk, 16))(keys)
    lines_t = lines.T.astype(jnp.int32)

    p2 = patches.reshape(t * b, c)
    out2, fwd, bwd = _shuffle_sc(p2, lines_t)
    return out2.reshape(REMAIN_T, b, c), fwd, bwd


# speedup vs baseline: 65.3300x; 1.1004x over previous
import functools

import jax
import jax.numpy as jnp
from jax import lax
from jax.experimental import pallas as pl
from jax.experimental.pallas import tpu as pltpu
from jax.experimental.pallas import tpu_sc as plsc

T = 320
B = 128
C = 768
REMAIN_T = 159
NC, NS, L = 2, 16, 16
NW = NC * NS
ROWS = 64
NG = REMAIN_T * 2
KMAX = (NG + NW - 1) // NW
FWD_PER_W = 16
NWF = T // FWD_PER_W


def _shuffle_sc(p2, lines_t):
    mesh = plsc.VectorSubcoreMesh(core_axis_name="c", subcore_axis_name="s")

    @functools.partial(
        pl.kernel,
        mesh=mesh,
        out_type=[
            jax.ShapeDtypeStruct((REMAIN_T * B, C), jnp.float32),
            jax.ShapeDtypeStruct((T, B), jnp.int32),
            jax.ShapeDtypeStruct((T, B), jnp.int32),
        ],
        scratch_types=[
            pltpu.VMEM((16, B), jnp.int32),
            pltpu.VMEM((ROWS,), jnp.int32),
            pltpu.VMEM((ROWS,), jnp.int32),
            pltpu.VMEM((ROWS, C), jnp.float32),
            pltpu.VMEM((ROWS, C), jnp.float32),
            pltpu.VMEM((FWD_PER_W, B), jnp.int32),
            pltpu.VMEM((FWD_PER_W, B), jnp.int32),
            pltpu.SemaphoreType.DMA,
            pltpu.SemaphoreType.DMA,
            pltpu.SemaphoreType.DMA,
            pltpu.SemaphoreType.DMA,
        ],
    )
    def k(p2_hbm, lines_hbm, out_hbm, fwd_hbm, bwd_hbm,
          lines_v, idx0, idx1, buf0, buf1, stf, stb,
          gsem0, gsem1, ssem0, ssem1):
        wid = lax.axis_index("s") * NC + lax.axis_index("c")
        iota = lax.iota(jnp.int32, L)
        zeros = jnp.zeros((L,), jnp.int32)

        idxs = (idx0, idx1)
        bufs = (buf0, buf1)
        gsems = (gsem0, gsem1)
        ssems = (ssem0, ssem1)

        pltpu.sync_copy(lines_hbm, lines_v)

        def fill_idx(kk):
            g = wid + NW * kk
            t = g // 2
            b0 = (g % 2) * ROWS
            jf = t // 20
            i_f = t % 20
            for c in range(ROWS // L):
                lvec = lines_v[jf, pl.ds(b0 + c * L, L)]
                bvec = b0 + c * L + iota
                idxs[kk & 1][pl.ds(c * L, L)] = 2048 * i_f + 128 * lvec + bvec

        def gather_desc(kk):
            i = kk & 1
            return pltpu.make_async_copy(p2_hbm.at[idxs[i]], bufs[i], gsems[i])

        def scatter_desc(kk):
            i = kk & 1
            g = wid + NW * kk
            return pltpu.make_async_copy(
                bufs[i], out_hbm.at[pl.ds(g * ROWS, ROWS)], ssems[i])

        fill_idx(0)
        gather_desc(0).start()

        @pl.when(wid < NWF)
        def _():
            k0 = wid * FWD_PER_W
            for c in range(B // L):
                p_lo = zeros
                p_hi = zeros
                for j in range(16):
                    lv = lines_v[j, pl.ds(c * L, L)]
                    amt = (lv & 7) << 2
                    sh = jnp.full((L,), j, jnp.int32) << amt
                    lo = lv < 8
                    p_lo = p_lo + jnp.where(lo, sh, zeros)
                    p_hi = p_hi + jnp.where(lo, zeros, sh)
                for rr in range(FWD_PER_W):
                    krow = k0 + rr
                    jf = krow // 20
                    i_f = krow % 20
                    p = p_lo if rr < 8 else p_hi
                    inv_vec = lax.shift_right_logical(
                        p, jnp.int32(4 * (rr & 7))) & 15
                    stf[rr, pl.ds(c * L, L)] = (
                        16 * i_f + lines_v[jf, pl.ds(c * L, L)])
                    stb[rr, pl.ds(c * L, L)] = 20 * inv_vec + wid
            pltpu.sync_copy(stf, fwd_hbm.at[pl.ds(k0, FWD_PER_W)])
            pltpu.sync_copy(stb, bwd_hbm.at[pl.ds(k0, FWD_PER_W)])

        g_last = wid + NW * (KMAX - 1)
        for kk in range(1, KMAX):
            if kk >= 2:
                scatter_desc(kk - 2).wait()
            if kk < KMAX - 1:
                fill_idx(kk)
                gather_desc(kk).start()
            else:
                @pl.when(g_last < NG)
                def _(kk=kk):
                    fill_idx(kk)
                    gather_desc(kk).start()
            gather_desc(kk - 1).wait()
            scatter_desc(kk - 1).start()

        @pl.when(g_last < NG)
        def _():
            gather_desc(KMAX - 1).wait()
            scatter_desc(KMAX - 1).start()

        scatter_desc(KMAX - 2).wait()

        @pl.when(g_last < NG)
        def _():
            scatter_desc(KMAX - 1).wait()

    return k(p2, lines_t)


def kernel(patches):
    t, b, c = patches.shape
    keys = jax.random.split(jax.random.key(42), b)
    lines = jax.vmap(lambda kk: jax.random.permutation(kk, 16))(keys)
    lines_t = lines.T.astype(jnp.int32)

    p2 = patches.reshape(t * b, c)
    out2, fwd, bwd = _shuffle_sc(p2, lines_t)
    return out2.reshape(REMAIN_T, b, c), fwd, bwd
